# Initial kernel scaffold; baseline (speedup 1.0000x reference)
import jax
import jax.numpy as jnp
from jax.experimental import pallas as pl


def kernel(embeds, centroids):
    raise NotImplementedError("write your pallas kernel here")


# speedup vs baseline: 133.1369x; 133.1369x over previous
import functools
import math

import jax
import jax.numpy as jnp
from jax.experimental import pallas as pl


def _tile_kernel(e_ref, ct_ref, out_ref):
    e = e_ref[...]
    ct = ct_ref[...]
    enorm = jnp.sum(e * e, axis=1, keepdims=True)
    cnorm = jnp.sum(ct * ct, axis=0, keepdims=True)
    dot = jnp.dot(e, ct, preferred_element_type=jnp.float32)
    dist = (enorm + cnorm) - 2.0 * dot
    out_ref[...] = jnp.sqrt(jnp.min(dist, axis=1, keepdims=True))


@functools.partial(jax.jit, static_argnames=("tile_m",))
def _min_dist(embeds_flat, centroids_t, tile_m):
    rows = embeds_flat.shape[0]
    d, p = centroids_t.shape
    grid = (rows // tile_m,)
    return pl.pallas_call(
        _tile_kernel,
        grid=grid,
        in_specs=[
            pl.BlockSpec((tile_m, d), lambda i: (i, 0)),
            pl.BlockSpec((d, p), lambda i: (0, 0)),
        ],
        out_specs=pl.BlockSpec((tile_m, 1), lambda i: (i, 0)),
        out_shape=jax.ShapeDtypeStruct((rows, 1), jnp.float32),
    )(embeds_flat, centroids_t)


def kernel(embeds, centroids):
    b, n, d = embeds.shape
    h = int(math.sqrt(n))
    score = _min_dist(embeds.reshape(b * n, d), centroids.T, 512)
    score = score.reshape(b, h, h, 1).transpose(0, 3, 1, 2)
    return (jnp.zeros(()), score)
